# Initial kernel scaffold; baseline (speedup 1.0000x reference)
import jax
import jax.numpy as jnp
from jax.experimental import pallas as pl


def kernel(depth, features, mask_logits, locations, intrinsics):
    raise NotImplementedError("write your pallas kernel here")


# speedup vs baseline: 4.7373x; 4.7373x over previous
import functools

import jax
import jax.numpy as jnp
from jax import lax
from jax.experimental import pallas as pl
from jax.experimental.pallas import tpu as pltpu
from jax.experimental.pallas import tpu_sc as plsc

IMG_H, IMG_W = 120, 160
TRUNC = 3
VOXEL = 0.05
DMIN, DMAX = 0.4, 6.0
MAX_INST = 16
FRUSTUM_DIMS = 256.0

HW = IMG_H * IMG_W
NREP = 2 * TRUNC + 1
NFEAT = 32
NCH = 2 + NFEAT + MAX_INST + 1
NSUB = 32
PIX_PER_SUB = HW * 2 // NSUB
CHUNK = 80
NCHUNK = PIX_PER_SUB // CHUNK
NGRP = CHUNK // 16
ROWS = CHUNK * NREP


def _rne_bf16(v):
    u = lax.bitcast_convert_type(v, jnp.uint32)
    bias = jnp.uint32(0x7FFF) + ((u >> jnp.uint32(16)) & jnp.uint32(1))
    return lax.bitcast_convert_type((u + bias) & jnp.uint32(0xFFFF0000),
                                    jnp.float32)


def _sc_body(depth_hbm, feat_hbm, mask_hbm, const_hbm, sel_hbm,
             coords_hbm, feats_hbm,
             depth_vm, feat_vm, mvm, const_vm, sel_vm, fout_vm, cout_vm):
    cid = lax.axis_index("c")
    sid = lax.axis_index("s")
    wid = sid * 2 + cid
    b = wid // 16
    lw = wid % 16

    pltpu.sync_copy(const_hbm.at[b], const_vm)
    pltpu.sync_copy(sel_hbm.at[b], sel_vm)

    iota16 = lax.iota(jnp.int32, 16)
    zeros16 = jnp.zeros((16,), jnp.float32)
    for q in range(NGRP):
        mvm[16, pl.ds(q * 16, 16)] = zeros16

    def csplat(j):
        return const_vm[j, :]

    def bsplat(j):
        return _rne_bf16(csplat(j))

    i00, i01, i02, i03 = bsplat(0), bsplat(1), bsplat(2), bsplat(3)
    i10, i11, i12, i13 = bsplat(4), bsplat(5), bsplat(6), bsplat(7)
    i20, i21, i22, i23 = bsplat(8), bsplat(9), bsplat(10), bsplat(11)
    tx, ty, tz = bsplat(12), bsplat(13), bsplat(14)
    padx, pady, padz = csplat(16), csplat(17), csplat(18)
    i30, i31, i32_, i33 = bsplat(20), bsplat(21), bsplat(22), bsplat(23)
    selsp = [sel_vm[k, :] for k in range(MAX_INST)]

    svox = jnp.full((16,), 1.0 / VOXEL, jnp.float32)
    bsp = jnp.zeros((16,), jnp.int32) + b
    io7 = iota16 * NREP

    def chunk_body(j, carry):
        p0 = lw * PIX_PER_SUB + j * CHUNK
        gp0 = b * HW + p0
        pltpu.sync_copy(depth_hbm.at[b, pl.ds(p0, CHUNK)], depth_vm)
        pltpu.sync_copy(feat_hbm.at[b, :, pl.ds(p0, CHUNK)], feat_vm)
        pltpu.sync_copy(mask_hbm.at[b, :, pl.ds(p0, CHUNK)],
                        mvm.at[pl.ds(0, 16), :])

        for g in range(NGRP):
            gp = g * 16
            z = depth_vm[pl.ds(gp, 16)]
            pv = (p0 + gp) + iota16
            yv = pv // IMG_W
            xv = pv - yv * IMG_W
            xz = _rne_bf16(xv.astype(jnp.float32) * z)
            yz = _rne_bf16(yv.astype(jnp.float32) * z)
            zb = _rne_bf16(z)
            pcx = _rne_bf16(i00 * xz + i01 * yz + i02 * zb + i03)
            pcy = _rne_bf16(i10 * xz + i11 * yz + i12 * zb + i13)
            pcz = _rne_bf16(i20 * xz + i21 * yz + i22 * zb + i23)
            pc3 = _rne_bf16(i30 * xz + i31 * yz + i32_ * zb + i33)
            gx = svox * pcx + tx * pc3
            gy = svox * pcy + ty * pc3
            gz = svox * pcz + tz * pc3
            cxi = (gx + padx).astype(jnp.int32)
            cyi = (gy + pady).astype(jnp.int32)
            gzp = gz + padz
            fz = gz - gz.astype(jnp.int32).astype(jnp.float32)

            row0 = io7 + gp * NREP
            for r in range(NREP):
                rows = row0 + r
                czr = (gzp + float(r - TRUNC)).astype(jnp.int32)
                plsc.store_scatter(cout_vm, [rows, jnp.full((16,), 0, jnp.int32)], bsp)
                plsc.store_scatter(cout_vm, [rows, jnp.full((16,), 1, jnp.int32)], cxi)
                plsc.store_scatter(cout_vm, [rows, jnp.full((16,), 2, jnp.int32)], cyi)
                plsc.store_scatter(cout_vm, [rows, jnp.full((16,), 3, jnp.int32)], czr)
                d = fz + float(r - TRUNC)
                plsc.store_scatter(fout_vm, [rows, jnp.full((16,), 0, jnp.int32)], jnp.sign(d))
                plsc.store_scatter(fout_vm, [rows, jnp.full((16,), 1, jnp.int32)], jnp.abs(d))

            for ch in range(NFEAT):
                v = feat_vm[ch, pl.ds(gp, 16)]
                col = jnp.full((16,), 2 + ch, jnp.int32)
                for r in range(NREP):
                    plsc.store_scatter(fout_vm, [row0 + r, col], v)

            acc = mvm[0, pl.ds(gp, 16)]
            for ci in range(1, MAX_INST):
                acc = acc + mvm[ci, pl.ds(gp, 16)]
            lab = jnp.where(acc == 0.0, jnp.full((16,), 1.0, jnp.float32),
                            jnp.full((16,), 0.0, jnp.float32))
            col = jnp.full((16,), 2 + NFEAT, jnp.int32)
            for r in range(NREP):
                plsc.store_scatter(fout_vm, [row0 + r, col], lab)

            colidx = gp + iota16
            for cc in range(MAX_INST):
                v = plsc.load_gather(mvm, [selsp[cc], colidx])
                col = jnp.full((16,), 3 + NFEAT + cc, jnp.int32)
                for r in range(NREP):
                    plsc.store_scatter(fout_vm, [row0 + r, col], v)

        pltpu.sync_copy(fout_vm, feats_hbm.at[pl.ds(gp0 * NREP, ROWS), :])
        pltpu.sync_copy(cout_vm, coords_hbm.at[pl.ds(gp0 * NREP, ROWS), :])
        return carry

    lax.fori_loop(0, NCHUNK, chunk_body, 0)


def kernel(depth, features, mask_logits, locations, intrinsics):
    B = depth.shape[0]
    f32 = jnp.float32

    intr_inv = jnp.stack([jnp.linalg.inv(intrinsics[i]) for i in range(B)])
    xs = jnp.array([0.0, IMG_W, 0.0, IMG_W, 0.0, IMG_W, 0.0, IMG_W], f32)
    ys = jnp.array([0.0, 0.0, IMG_H, IMG_H, 0.0, 0.0, IMG_H, IMG_H], f32)
    zs = jnp.array([DMIN] * 4 + [DMAX] * 4, f32)
    pix = jnp.stack([xs * zs, ys * zs, zs, jnp.ones(8, f32)], axis=0)
    pts = intr_inv @ pix
    mn = jnp.min(pts[:, :3], axis=2)
    mx = jnp.max(pts[:, :3], axis=2)
    t = -mn / VOXEL
    dims = jnp.floor((mx - mn) / VOXEL) + 1.0
    pad = jnp.floor((FRUSTUM_DIMS - dims) / 2.0)

    zcol = jnp.zeros((B, 1), f32)
    const = jnp.concatenate([
        intr_inv[:, 0, :], intr_inv[:, 1, :], intr_inv[:, 2, :],
        t, zcol,
        pad, zcol,
        intr_inv[:, 3, :],
        jnp.zeros((B, 8), f32),
    ], axis=1).astype(f32)
    const = jnp.tile(const[:, :, None], (1, 1, 16))

    ar = jnp.arange(MAX_INST, dtype=jnp.int32)
    sel_cols = []
    for c in range(MAX_INST):
        li = jnp.max(jnp.where(locations == c, ar[None, :], -1), axis=1)
        sel_cols.append(jnp.where(li < 0, MAX_INST, li))
    sel = jnp.stack(sel_cols, axis=1).astype(jnp.int32)
    sel = jnp.tile(sel[:, :, None], (1, 1, 16))

    depth2 = depth.reshape(B, HW)
    feat3 = features.reshape(B, NFEAT, HW)
    mask3 = mask_logits.reshape(B, MAX_INST, HW)

    nrows = B * HW * NREP
    mesh = plsc.VectorSubcoreMesh(core_axis_name="c", subcore_axis_name="s",
                                  num_cores=2, num_subcores=16)
    run = pl.kernel(
        _sc_body,
        out_type=(
            jax.ShapeDtypeStruct((nrows, 4), jnp.int32),
            jax.ShapeDtypeStruct((nrows, NCH), jnp.float32),
        ),
        mesh=mesh,
        scratch_types=[
            pltpu.VMEM((CHUNK,), jnp.float32),
            pltpu.VMEM((NFEAT, CHUNK), jnp.float32),
            pltpu.VMEM((MAX_INST + 1, CHUNK), jnp.float32),
            pltpu.VMEM((32, 16), jnp.float32),
            pltpu.VMEM((MAX_INST, 16), jnp.int32),
            pltpu.VMEM((ROWS, NCH), jnp.float32),
            pltpu.VMEM((ROWS, 4), jnp.int32),
        ],
        compiler_params=pltpu.CompilerParams(
            use_tc_tiling_on_sc=False, needs_layout_passes=False),
    )
    coords, feats = run(depth2, feat3, mask3, const, sel)
    return coords, feats


# speedup vs baseline: 4.9675x; 1.0486x over previous
import functools

import jax
import jax.numpy as jnp
from jax import lax
from jax.experimental import pallas as pl
from jax.experimental.pallas import tpu as pltpu
from jax.experimental.pallas import tpu_sc as plsc

IMG_H, IMG_W = 120, 160
TRUNC = 3
VOXEL = 0.05
DMIN, DMAX = 0.4, 6.0
MAX_INST = 16
FRUSTUM_DIMS = 256.0

HW = IMG_H * IMG_W
NREP = 2 * TRUNC + 1
NFEAT = 32
NCH = 2 + NFEAT + MAX_INST + 1
NSUB = 32
PIX_PER_SUB = HW * 2 // NSUB
CHUNK = 80
NCHUNK = PIX_PER_SUB // CHUNK
NGRP = CHUNK // 16
ROWS = CHUNK * NREP


def _rne_bf16(v):
    u = lax.bitcast_convert_type(v, jnp.uint32)
    bias = jnp.uint32(0x7FFF) + ((u >> jnp.uint32(16)) & jnp.uint32(1))
    return lax.bitcast_convert_type((u + bias) & jnp.uint32(0xFFFF0000),
                                    jnp.float32)


def _sc_body(depth_hbm, feat_hbm, mask_hbm, const_hbm, sel_hbm,
             coords_hbm, feats_hbm,
             depth_vm, feat_vm, mvm, const_vm, sel_vm, fout_vm, cout_vm,
             isem, osem):
    cid = lax.axis_index("c")
    sid = lax.axis_index("s")
    wid = sid * 2 + cid
    b = wid // 16
    lw = wid % 16

    pltpu.sync_copy(const_hbm.at[b], const_vm)
    pltpu.sync_copy(sel_hbm.at[b], sel_vm)

    iota16 = lax.iota(jnp.int32, 16)
    zeros16 = jnp.zeros((16,), jnp.float32)
    for q in range(NGRP):
        mvm[16, pl.ds(q * 16, 16)] = zeros16

    def csplat(j):
        return const_vm[j, :]

    def bsplat(j):
        return _rne_bf16(csplat(j))

    i00, i01, i02, i03 = bsplat(0), bsplat(1), bsplat(2), bsplat(3)
    i10, i11, i12, i13 = bsplat(4), bsplat(5), bsplat(6), bsplat(7)
    i20, i21, i22, i23 = bsplat(8), bsplat(9), bsplat(10), bsplat(11)
    tx, ty, tz = bsplat(12), bsplat(13), bsplat(14)
    padx, pady, padz = csplat(16), csplat(17), csplat(18)
    i30, i31, i32_, i33 = bsplat(20), bsplat(21), bsplat(22), bsplat(23)
    selsp = [sel_vm[k, :] for k in range(MAX_INST)]

    svox = jnp.full((16,), 1.0 / VOXEL, jnp.float32)
    bsp = jnp.zeros((16,), jnp.int32) + b
    io7 = iota16 * NREP

    def chunk_body(j, carry):
        p0 = lw * PIX_PER_SUB + j * CHUNK
        gp0 = b * HW + p0
        buf = lax.rem(j, 2)
        boff = buf * ROWS
        fsrc = fout_vm.at[pl.ds(boff, ROWS), :]
        csrc = cout_vm.at[pl.ds(boff, ROWS), :]
        fdst = feats_hbm.at[pl.ds(gp0 * NREP, ROWS), :]
        cdst = coords_hbm.at[pl.ds(gp0 * NREP, ROWS), :]

        h1 = pltpu.async_copy(depth_hbm.at[b, pl.ds(p0, CHUNK)], depth_vm, isem)
        h2 = pltpu.async_copy(feat_hbm.at[b, :, pl.ds(p0, CHUNK)], feat_vm, isem)
        h3 = pltpu.async_copy(mask_hbm.at[b, :, pl.ds(p0, CHUNK)],
                              mvm.at[pl.ds(0, 16), :], isem)

        @pl.when(j >= 2)
        def _drain():
            pltpu.make_async_copy(fsrc, fdst, osem).wait()
            pltpu.make_async_copy(csrc, cdst, osem).wait()

        h1.wait()
        h2.wait()
        h3.wait()

        for g in range(NGRP):
            gp = g * 16
            z = depth_vm[pl.ds(gp, 16)]
            pv = (p0 + gp) + iota16
            yv = pv // IMG_W
            xv = pv - yv * IMG_W
            xz = _rne_bf16(xv.astype(jnp.float32) * z)
            yz = _rne_bf16(yv.astype(jnp.float32) * z)
            zb = _rne_bf16(z)
            pcx = _rne_bf16(i00 * xz + i01 * yz + i02 * zb + i03)
            pcy = _rne_bf16(i10 * xz + i11 * yz + i12 * zb + i13)
            pcz = _rne_bf16(i20 * xz + i21 * yz + i22 * zb + i23)
            pc3 = _rne_bf16(i30 * xz + i31 * yz + i32_ * zb + i33)
            gx = svox * pcx + tx * pc3
            gy = svox * pcy + ty * pc3
            gz = svox * pcz + tz * pc3
            cxi = (gx + padx).astype(jnp.int32)
            cyi = (gy + pady).astype(jnp.int32)
            gzp = gz + padz
            fz = gz - gz.astype(jnp.int32).astype(jnp.float32)

            row0 = io7 + gp * NREP + boff
            for r in range(NREP):
                rows = row0 + r
                czr = (gzp + float(r - TRUNC)).astype(jnp.int32)
                plsc.store_scatter(cout_vm, [rows, jnp.full((16,), 0, jnp.int32)], bsp)
                plsc.store_scatter(cout_vm, [rows, jnp.full((16,), 1, jnp.int32)], cxi)
                plsc.store_scatter(cout_vm, [rows, jnp.full((16,), 2, jnp.int32)], cyi)
                plsc.store_scatter(cout_vm, [rows, jnp.full((16,), 3, jnp.int32)], czr)
                d = fz + float(r - TRUNC)
                plsc.store_scatter(fout_vm, [rows, jnp.full((16,), 0, jnp.int32)], jnp.sign(d))
                plsc.store_scatter(fout_vm, [rows, jnp.full((16,), 1, jnp.int32)], jnp.abs(d))

            for ch in range(NFEAT):
                v = feat_vm[ch, pl.ds(gp, 16)]
                col = jnp.full((16,), 2 + ch, jnp.int32)
                for r in range(NREP):
                    plsc.store_scatter(fout_vm, [row0 + r, col], v)

            acc = mvm[0, pl.ds(gp, 16)]
            for ci in range(1, MAX_INST):
                acc = acc + mvm[ci, pl.ds(gp, 16)]
            lab = jnp.where(acc == 0.0, jnp.full((16,), 1.0, jnp.float32),
                            jnp.full((16,), 0.0, jnp.float32))
            col = jnp.full((16,), 2 + NFEAT, jnp.int32)
            for r in range(NREP):
                plsc.store_scatter(fout_vm, [row0 + r, col], lab)

            colidx = gp + iota16
            for cc in range(MAX_INST):
                v = plsc.load_gather(mvm, [selsp[cc], colidx])
                col = jnp.full((16,), 3 + NFEAT + cc, jnp.int32)
                for r in range(NREP):
                    plsc.store_scatter(fout_vm, [row0 + r, col], v)

        pltpu.async_copy(fsrc, fdst, osem)
        pltpu.async_copy(csrc, cdst, osem)
        return carry

    lax.fori_loop(0, NCHUNK, chunk_body, 0)
    for _ in range(2):
        pltpu.make_async_copy(fout_vm.at[pl.ds(0, ROWS), :],
                              feats_hbm.at[pl.ds(0, ROWS), :], osem).wait()
        pltpu.make_async_copy(cout_vm.at[pl.ds(0, ROWS), :],
                              coords_hbm.at[pl.ds(0, ROWS), :], osem).wait()


def kernel(depth, features, mask_logits, locations, intrinsics):
    B = depth.shape[0]
    f32 = jnp.float32

    intr_inv = jnp.stack([jnp.linalg.inv(intrinsics[i]) for i in range(B)])
    xs = jnp.array([0.0, IMG_W, 0.0, IMG_W, 0.0, IMG_W, 0.0, IMG_W], f32)
    ys = jnp.array([0.0, 0.0, IMG_H, IMG_H, 0.0, 0.0, IMG_H, IMG_H], f32)
    zs = jnp.array([DMIN] * 4 + [DMAX] * 4, f32)
    pix = jnp.stack([xs * zs, ys * zs, zs, jnp.ones(8, f32)], axis=0)
    pts = intr_inv @ pix
    mn = jnp.min(pts[:, :3], axis=2)
    mx = jnp.max(pts[:, :3], axis=2)
    t = -mn / VOXEL
    dims = jnp.floor((mx - mn) / VOXEL) + 1.0
    pad = jnp.floor((FRUSTUM_DIMS - dims) / 2.0)

    zcol = jnp.zeros((B, 1), f32)
    const = jnp.concatenate([
        intr_inv[:, 0, :], intr_inv[:, 1, :], intr_inv[:, 2, :],
        t, zcol,
        pad, zcol,
        intr_inv[:, 3, :],
        jnp.zeros((B, 8), f32),
    ], axis=1).astype(f32)
    const = jnp.tile(const[:, :, None], (1, 1, 16))

    ar = jnp.arange(MAX_INST, dtype=jnp.int32)
    sel_cols = []
    for c in range(MAX_INST):
        li = jnp.max(jnp.where(locations == c, ar[None, :], -1), axis=1)
        sel_cols.append(jnp.where(li < 0, MAX_INST, li))
    sel = jnp.stack(sel_cols, axis=1).astype(jnp.int32)
    sel = jnp.tile(sel[:, :, None], (1, 1, 16))

    depth2 = depth.reshape(B, HW)
    feat3 = features.reshape(B, NFEAT, HW)
    mask3 = mask_logits.reshape(B, MAX_INST, HW)

    nrows = B * HW * NREP
    mesh = plsc.VectorSubcoreMesh(core_axis_name="c", subcore_axis_name="s",
                                  num_cores=2, num_subcores=16)
    run = pl.kernel(
        _sc_body,
        out_type=(
            jax.ShapeDtypeStruct((nrows, 4), jnp.int32),
            jax.ShapeDtypeStruct((nrows, NCH), jnp.float32),
        ),
        mesh=mesh,
        scratch_types=[
            pltpu.VMEM((CHUNK,), jnp.float32),
            pltpu.VMEM((NFEAT, CHUNK), jnp.float32),
            pltpu.VMEM((MAX_INST + 1, CHUNK), jnp.float32),
            pltpu.VMEM((32, 16), jnp.float32),
            pltpu.VMEM((MAX_INST, 16), jnp.int32),
            pltpu.VMEM((2 * ROWS, NCH), jnp.float32),
            pltpu.VMEM((2 * ROWS, 4), jnp.int32),
            pltpu.SemaphoreType.DMA,
            pltpu.SemaphoreType.DMA,
        ],
        compiler_params=pltpu.CompilerParams(
            use_tc_tiling_on_sc=False, needs_layout_passes=False),
    )
    coords, feats = run(depth2, feat3, mask3, const, sel)
    return coords, feats
